# baseline (speedup 1.0000x reference)
import functools

import jax
import jax.numpy as jnp
from jax.experimental import pallas as pl
from jax.experimental.pallas import tpu as pltpu

_ROW_TILE = 128
_CHUNK = 128


def _routing_kernel(ei_ref, pos_ref, te_ref, na_ref, *, e_num, t_tile, nt_pad):
    nc, cl = ei_ref.shape
    ei = ei_ref[...]
    li = jax.lax.broadcasted_iota(jnp.int32, (cl, cl), 0)
    lj = jax.lax.broadcasted_iota(jnp.int32, (cl, cl), 1)
    tri_incl = (li <= lj).astype(jnp.float32)
    ri = jax.lax.broadcasted_iota(jnp.int32, (nc, nc), 0)
    rj = jax.lax.broadcasted_iota(jnp.int32, (nc, nc), 1)
    tri_excl_rows = (ri > rj).astype(jnp.float32)

    pos = jnp.zeros((nc, cl), jnp.int32)
    counts = []
    ranks = []
    for e in range(e_num):
        m = (ei == e).astype(jnp.float32)
        cum = jax.lax.dot_general(m, tri_incl, (((1,), (0,)), ((), ())),
                                  preferred_element_type=jnp.float32)
        tot = cum[:, cl - 1:cl]
        prefrow = jax.lax.dot_general(tri_excl_rows, tot, (((1,), (0,)), ((), ())),
                                      preferred_element_type=jnp.float32)
        rank = (prefrow + cum - 1.0).astype(jnp.int32)
        ranks.append((m, rank))
        counts.append(jnp.sum(tot).astype(jnp.int32))

    nact = jnp.int32(0)
    tile_base = []
    for e in range(e_num):
        tile_base.append(nact)
        nact = nact + (counts[e] + t_tile - 1) // t_tile
    for e in range(e_num):
        m, rank = ranks[e]
        seg = tile_base[e] * t_tile
        pos = pos + m.astype(jnp.int32) * (seg + rank)
    pos_ref[...] = pos

    tl = jax.lax.broadcasted_iota(jnp.int32, (1, nt_pad), 1)
    te = jnp.full((1, nt_pad), -1, jnp.int32)
    for e in range(e_num):
        te = te + (tl >= tile_base[e]).astype(jnp.int32)
    te_ref[...] = te
    na_ref[...] = jnp.full((1, 1), nact, jnp.int32)


def _gate_up_kernel(te_ref, na_ref, x_ref, wg_ref, wu_ref, pos_ref, h_ref,
                    *, t_tile):
    t = pl.program_id(0)

    @pl.when(t < na_ref[0])
    def _active():
        base = t * t_tile
        rowids = base + jax.lax.broadcasted_iota(jnp.int32, (t_tile, 1), 0)
        gmat = ((pos_ref[0:1, :] == rowids).astype(jnp.bfloat16)
                + (pos_ref[1:2, :] == rowids).astype(jnp.bfloat16))
        xs = jax.lax.dot_general(gmat, x_ref[...].astype(jnp.bfloat16),
                                 (((1,), (0,)), ((), ())),
                                 preferred_element_type=jnp.float32)
        g = jax.lax.dot_general(xs, wg_ref[0], (((1,), (1,)), ((), ())),
                                preferred_element_type=jnp.float32)
        u = jax.lax.dot_general(xs, wu_ref[0], (((1,), (1,)), ((), ())),
                                preferred_element_type=jnp.float32)
        h_ref[...] = ((g * jax.lax.logistic(g)) * u).astype(jnp.bfloat16)


def _down_kernel(te_ref, na_ref, h_ref, wd_ref, pos_ref, w_ref, y_ref,
                 *, t_tile):
    t = pl.program_id(0)

    @pl.when(t == 0)
    def _init():
        y_ref[...] = jnp.zeros_like(y_ref)

    @pl.when(t < na_ref[0])
    def _active():
        base = t * t_tile
        rowids = base + jax.lax.broadcasted_iota(jnp.int32, (t_tile, 1), 0)
        out = jax.lax.dot_general(h_ref[...], wd_ref[0].astype(jnp.bfloat16),
                                  (((1,), (1,)), ((), ())),
                                  preferred_element_type=jnp.float32)
        c0 = (pos_ref[0:1, :] == rowids).astype(jnp.float32)
        c1 = (pos_ref[1:2, :] == rowids).astype(jnp.float32)
        smat = (w_ref[0:1, :] * c0 + w_ref[1:2, :] * c1).astype(jnp.bfloat16)
        y_ref[...] += jax.lax.dot_general(smat, out.astype(jnp.bfloat16),
                                          (((0,), (0,)), ((), ())),
                                          preferred_element_type=jnp.float32)


def kernel(x, expert_weights, expert_indices, top_k, Wg, Wu, Wd):
    n, h_dim = x.shape
    k = expert_indices.shape[1]
    e_num, i_dim, _ = Wg.shape
    s = n * k
    t_tile = _ROW_TILE
    nt = s // t_tile + (e_num - 1)
    nt_pad = ((nt + 127) // 128) * 128
    p = nt * t_tile

    flat2 = expert_indices.reshape(s // _CHUNK, _CHUNK).astype(jnp.int32)
    pos2, te2, na2 = pl.pallas_call(
        functools.partial(_routing_kernel, e_num=e_num, t_tile=t_tile,
                          nt_pad=nt_pad),
        out_shape=(
            jax.ShapeDtypeStruct((s // _CHUNK, _CHUNK), jnp.int32),
            jax.ShapeDtypeStruct((1, nt_pad), jnp.int32),
            jax.ShapeDtypeStruct((1, 1), jnp.int32),
        ),
    )(flat2)
    tile_expert = te2[0, :nt]
    nact = na2.reshape(1)
    pos_t = pos2.reshape(n, k).T
    w_t = (expert_weights * jnp.equal(top_k, k).astype(expert_weights.dtype)).T

    gs1 = pltpu.PrefetchScalarGridSpec(
        num_scalar_prefetch=2,
        grid=(nt,),
        in_specs=[
            pl.BlockSpec((n, h_dim), lambda t, te, na: (0, 0)),
            pl.BlockSpec((1, i_dim, h_dim), lambda t, te, na: (te[t], 0, 0)),
            pl.BlockSpec((1, i_dim, h_dim), lambda t, te, na: (te[t], 0, 0)),
            pl.BlockSpec((k, n), lambda t, te, na: (0, 0)),
        ],
        out_specs=pl.BlockSpec((t_tile, i_dim), lambda t, te, na: (t, 0)),
    )
    h_s = pl.pallas_call(
        functools.partial(_gate_up_kernel, t_tile=t_tile),
        grid_spec=gs1,
        out_shape=jax.ShapeDtypeStruct((p, i_dim), jnp.bfloat16),
        compiler_params=pltpu.CompilerParams(
            vmem_limit_bytes=63 * 1024 * 1024),
    )(tile_expert, nact, x, Wg, Wu, pos_t)

    gs2 = pltpu.PrefetchScalarGridSpec(
        num_scalar_prefetch=2,
        grid=(nt,),
        in_specs=[
            pl.BlockSpec((t_tile, i_dim), lambda t, te, na: (t, 0)),
            pl.BlockSpec((1, h_dim, i_dim), lambda t, te, na: (te[t], 0, 0)),
            pl.BlockSpec((k, n), lambda t, te, na: (0, 0)),
            pl.BlockSpec((k, n), lambda t, te, na: (0, 0)),
        ],
        out_specs=pl.BlockSpec((n, h_dim), lambda t, te, na: (0, 0)),
    )
    y = pl.pallas_call(
        functools.partial(_down_kernel, t_tile=t_tile),
        grid_spec=gs2,
        out_shape=jax.ShapeDtypeStruct((n, h_dim), jnp.float32),
        compiler_params=pltpu.CompilerParams(
            vmem_limit_bytes=63 * 1024 * 1024),
    )(tile_expert, nact, h_s, Wd, pos_t, w_t)
    return y


# speedup vs baseline: 1.4735x; 1.4735x over previous
import functools

import jax
import jax.numpy as jnp
from jax.experimental import pallas as pl
from jax.experimental.pallas import tpu as pltpu

_ROW_TILE = 256
_CHUNK = 128


def _routing_kernel(ei_ref, pos_ref, te_ref, na_ref, *, e_num, t_tile, nt_pad):
    nc, cl = ei_ref.shape
    ei = ei_ref[...]
    li = jax.lax.broadcasted_iota(jnp.int32, (cl, cl), 0)
    lj = jax.lax.broadcasted_iota(jnp.int32, (cl, cl), 1)
    tri_incl = (li <= lj).astype(jnp.float32)
    ri = jax.lax.broadcasted_iota(jnp.int32, (nc, nc), 0)
    rj = jax.lax.broadcasted_iota(jnp.int32, (nc, nc), 1)
    tri_excl_rows = (ri > rj).astype(jnp.float32)

    pos = jnp.zeros((nc, cl), jnp.int32)
    counts = []
    ranks = []
    for e in range(e_num):
        m = (ei == e).astype(jnp.float32)
        cum = jax.lax.dot_general(m, tri_incl, (((1,), (0,)), ((), ())),
                                  preferred_element_type=jnp.float32)
        tot = cum[:, cl - 1:cl]
        prefrow = jax.lax.dot_general(tri_excl_rows, tot, (((1,), (0,)), ((), ())),
                                      preferred_element_type=jnp.float32)
        rank = (prefrow + cum - 1.0).astype(jnp.int32)
        ranks.append((m, rank))
        counts.append(jnp.sum(tot).astype(jnp.int32))

    nact = jnp.int32(0)
    tile_base = []
    for e in range(e_num):
        tile_base.append(nact)
        nact = nact + (counts[e] + t_tile - 1) // t_tile
    for e in range(e_num):
        m, rank = ranks[e]
        seg = tile_base[e] * t_tile
        pos = pos + m.astype(jnp.int32) * (seg + rank)
    pos_ref[...] = pos

    tl = jax.lax.broadcasted_iota(jnp.int32, (1, nt_pad), 1)
    te = jnp.full((1, nt_pad), -1, jnp.int32)
    for e in range(e_num):
        te = te + (tl >= tile_base[e]).astype(jnp.int32)
    te_ref[...] = te
    na_ref[...] = jnp.full((1, 1), nact, jnp.int32)


def _gate_up_kernel(te_ref, na_ref, x_ref, wg_ref, wu_ref, pos_ref, h_ref,
                    *, t_tile):
    t = pl.program_id(0)

    @pl.when(t < na_ref[0])
    def _active():
        base = t * t_tile
        rowids = base + jax.lax.broadcasted_iota(jnp.int32, (t_tile, 1), 0)
        gmat = ((pos_ref[0:1, :] == rowids).astype(jnp.bfloat16)
                + (pos_ref[1:2, :] == rowids).astype(jnp.bfloat16))
        xs = jax.lax.dot_general(gmat, x_ref[...].astype(jnp.bfloat16),
                                 (((1,), (0,)), ((), ())),
                                 preferred_element_type=jnp.float32)
        g = jax.lax.dot_general(xs, wg_ref[0], (((1,), (1,)), ((), ())),
                                preferred_element_type=jnp.float32)
        u = jax.lax.dot_general(xs, wu_ref[0], (((1,), (1,)), ((), ())),
                                preferred_element_type=jnp.float32)
        h_ref[...] = ((g * jax.lax.logistic(g)) * u).astype(jnp.bfloat16)


def _down_kernel(te_ref, na_ref, h_ref, wd_ref, pos_ref, w_ref, y_ref,
                 *, t_tile):
    t = pl.program_id(0)

    @pl.when(t == 0)
    def _init():
        y_ref[...] = jnp.zeros_like(y_ref)

    @pl.when(t < na_ref[0])
    def _active():
        base = t * t_tile
        rowids = base + jax.lax.broadcasted_iota(jnp.int32, (t_tile, 1), 0)
        out = jax.lax.dot_general(h_ref[...], wd_ref[0].astype(jnp.bfloat16),
                                  (((1,), (1,)), ((), ())),
                                  preferred_element_type=jnp.float32)
        c0 = (pos_ref[0:1, :] == rowids).astype(jnp.float32)
        c1 = (pos_ref[1:2, :] == rowids).astype(jnp.float32)
        smat = (w_ref[0:1, :] * c0 + w_ref[1:2, :] * c1).astype(jnp.bfloat16)
        y_ref[...] += jax.lax.dot_general(smat, out.astype(jnp.bfloat16),
                                          (((0,), (0,)), ((), ())),
                                          preferred_element_type=jnp.float32)


def kernel(x, expert_weights, expert_indices, top_k, Wg, Wu, Wd):
    n, h_dim = x.shape
    k = expert_indices.shape[1]
    e_num, i_dim, _ = Wg.shape
    s = n * k
    t_tile = _ROW_TILE
    nt = s // t_tile + (e_num - 1)
    nt_pad = ((nt + 127) // 128) * 128
    p = nt * t_tile

    flat2 = expert_indices.reshape(s // _CHUNK, _CHUNK).astype(jnp.int32)
    pos2, te2, na2 = pl.pallas_call(
        functools.partial(_routing_kernel, e_num=e_num, t_tile=t_tile,
                          nt_pad=nt_pad),
        out_shape=(
            jax.ShapeDtypeStruct((s // _CHUNK, _CHUNK), jnp.int32),
            jax.ShapeDtypeStruct((1, nt_pad), jnp.int32),
            jax.ShapeDtypeStruct((1, 1), jnp.int32),
        ),
    )(flat2)
    tile_expert = te2[0, :nt]
    nact = na2.reshape(1)
    pos_t = pos2.reshape(n, k).T
    w_t = (expert_weights * jnp.equal(top_k, k).astype(expert_weights.dtype)).T

    gs1 = pltpu.PrefetchScalarGridSpec(
        num_scalar_prefetch=2,
        grid=(nt,),
        in_specs=[
            pl.BlockSpec((n, h_dim), lambda t, te, na: (0, 0)),
            pl.BlockSpec((1, i_dim, h_dim), lambda t, te, na: (te[t], 0, 0)),
            pl.BlockSpec((1, i_dim, h_dim), lambda t, te, na: (te[t], 0, 0)),
            pl.BlockSpec((k, n), lambda t, te, na: (0, 0)),
        ],
        out_specs=pl.BlockSpec((t_tile, i_dim), lambda t, te, na: (t, 0)),
    )
    h_s = pl.pallas_call(
        functools.partial(_gate_up_kernel, t_tile=t_tile),
        grid_spec=gs1,
        out_shape=jax.ShapeDtypeStruct((p, i_dim), jnp.bfloat16),
        compiler_params=pltpu.CompilerParams(
            vmem_limit_bytes=63 * 1024 * 1024),
    )(tile_expert, nact, x, Wg, Wu, pos_t)

    gs2 = pltpu.PrefetchScalarGridSpec(
        num_scalar_prefetch=2,
        grid=(nt,),
        in_specs=[
            pl.BlockSpec((t_tile, i_dim), lambda t, te, na: (t, 0)),
            pl.BlockSpec((1, h_dim, i_dim), lambda t, te, na: (te[t], 0, 0)),
            pl.BlockSpec((k, n), lambda t, te, na: (0, 0)),
            pl.BlockSpec((k, n), lambda t, te, na: (0, 0)),
        ],
        out_specs=pl.BlockSpec((n, h_dim), lambda t, te, na: (0, 0)),
    )
    y = pl.pallas_call(
        functools.partial(_down_kernel, t_tile=t_tile),
        grid_spec=gs2,
        out_shape=jax.ShapeDtypeStruct((n, h_dim), jnp.float32),
        compiler_params=pltpu.CompilerParams(
            vmem_limit_bytes=63 * 1024 * 1024),
    )(tile_expert, nact, h_s, Wd, pos_t, w_t)
    return y
